# baseline (speedup 1.0000x reference)
import functools
import math

import jax
import jax.numpy as jnp
from jax import lax
from jax.experimental import pallas as pl
from jax.experimental.pallas import tpu as pltpu
from jax.experimental.pallas import tpu_sc as plsc


def _gather_rows_sc(table, idx):
    n_rows = idx.shape[0]
    _, d = table.shape
    info = plsc.get_sparse_core_info()
    nw = info.num_cores * info.num_subcores
    b_per_w = n_rows // nw
    mesh = plsc.VectorSubcoreMesh(core_axis_name="c", subcore_axis_name="s")

    @functools.partial(
        pl.kernel,
        mesh=mesh,
        out_type=jax.ShapeDtypeStruct((n_rows, d), table.dtype),
        scratch_types=[
            pltpu.VMEM((b_per_w,), jnp.int32),
            pltpu.VMEM((b_per_w, d), table.dtype),
            pltpu.SemaphoreType.DMA,
        ],
    )
    def gather_kernel(table_hbm, idx_hbm, out_hbm, idx_v, rows_v, sem):
        wid = lax.axis_index("s") * info.num_cores + lax.axis_index("c")
        base = wid * b_per_w
        pltpu.sync_copy(idx_hbm.at[pl.ds(base, b_per_w)], idx_v)
        pltpu.async_copy(table_hbm.at[idx_v], rows_v, sem).wait()
        pltpu.sync_copy(rows_v, out_hbm.at[pl.ds(base, b_per_w)])

    return gather_kernel(table, idx)


def _matmul_body(x_ref, pos_ref, w_hbm, b_ref, o_ref, w_vmem, sem, *, scale, bn):
    first = ((pl.program_id(0) == 0) & (pl.program_id(1) == 0)
             & (pl.program_id(2) == 0))

    @pl.when(first)
    def _():
        cp = pltpu.make_async_copy(w_hbm, w_vmem.at[pl.ds(0, w_hbm.shape[0])], sem)
        cp.start()
        cp.wait()

    n = pl.program_id(2)
    xb = (x_ref[0, 0] * scale + pos_ref[0]).astype(jnp.bfloat16)
    acc = lax.dot_general(
        xb, w_vmem[pl.ds(n * bn, bn)], (((1,), (1,)), ((), ())),
        preferred_element_type=jnp.float32,
    )
    o_ref[0, 0] = acc + b_ref[...]


def kernel(src, tgt, emb, pos_src, pos_tgt, Wc, bc):
    b, l = src.shape
    v, d = emb.shape
    m = 2 * b * l
    bm = l

    idx = jnp.concatenate([src.reshape(-1), tgt.reshape(-1)]).astype(jnp.int32)
    x = _gather_rows_sc(emb, idx)

    pos_cat = jnp.stack([pos_src[:l, :d], pos_tgt[:l, :d]])
    w_bf = Wc.astype(jnp.bfloat16)
    bc2 = bc.reshape(1, v)

    x4 = x.reshape(2, b, l, d)
    bn = 1024
    nb = (v + bn - 1) // bn
    v_pad = nb * bn
    out = pl.pallas_call(
        functools.partial(_matmul_body, scale=math.sqrt(d), bn=bn),
        grid=(2, b, nb),
        in_specs=[
            pl.BlockSpec((1, 1, bm, d), lambda s, i, n: (s, i, 0, 0)),
            pl.BlockSpec((1, bm, d), lambda s, i, n: (s, 0, 0)),
            pl.BlockSpec(memory_space=pl.ANY),
            pl.BlockSpec((1, bn), lambda s, i, n: (0, n)),
        ],
        out_specs=pl.BlockSpec((1, 1, bm, bn), lambda s, i, n: (s, i, 0, n)),
        out_shape=jax.ShapeDtypeStruct((2, b, l, v), jnp.float32),
        scratch_shapes=[
            pltpu.VMEM((v_pad, d), jnp.bfloat16),
            pltpu.SemaphoreType.DMA,
        ],
        compiler_params=pltpu.CompilerParams(
            dimension_semantics=("arbitrary", "arbitrary", "arbitrary"),
        ),
    )(x4, pos_cat, w_bf, bc2)

    return out


# speedup vs baseline: 1.5075x; 1.5075x over previous
import functools
import math

import jax
import jax.numpy as jnp
from jax import lax
from jax.experimental import pallas as pl
from jax.experimental.pallas import tpu as pltpu
from jax.experimental.pallas import tpu_sc as plsc


def _gather_rows_sc(table, idx):
    n_rows = idx.shape[0]
    _, d = table.shape
    info = plsc.get_sparse_core_info()
    nw = info.num_cores * info.num_subcores
    b_per_w = n_rows // nw
    mesh = plsc.VectorSubcoreMesh(core_axis_name="c", subcore_axis_name="s")

    @functools.partial(
        pl.kernel,
        mesh=mesh,
        out_type=jax.ShapeDtypeStruct((n_rows, d), table.dtype),
        scratch_types=[
            pltpu.VMEM((b_per_w,), jnp.int32),
            pltpu.VMEM((b_per_w, d), table.dtype),
            pltpu.SemaphoreType.DMA,
        ],
    )
    def gather_kernel(table_hbm, idx_hbm, out_hbm, idx_v, rows_v, sem):
        wid = lax.axis_index("s") * info.num_cores + lax.axis_index("c")
        base = wid * b_per_w
        pltpu.sync_copy(idx_hbm.at[pl.ds(base, b_per_w)], idx_v)
        pltpu.async_copy(table_hbm.at[idx_v], rows_v, sem).wait()
        pltpu.sync_copy(rows_v, out_hbm.at[pl.ds(base, b_per_w)])

    return gather_kernel(table, idx)


def _matmul_body(x_ref, pos_ref, w_hbm, b_ref, o_ref, w_vmem, sem, *, scale, bn):
    first = ((pl.program_id(0) == 0) & (pl.program_id(1) == 0)
             & (pl.program_id(2) == 0))

    @pl.when(first)
    def _():
        cp = pltpu.make_async_copy(w_hbm, w_vmem.at[pl.ds(0, w_hbm.shape[0])], sem)
        cp.start()
        cp.wait()

    n = pl.program_id(2)
    xb = (x_ref[0, 0] * scale + pos_ref[0]).astype(jnp.bfloat16)
    acc = lax.dot_general(
        w_vmem[pl.ds(n * bn, bn)], xb, (((1,), (1,)), ((), ())),
        preferred_element_type=jnp.float32,
    )
    o_ref[0, 0] = acc + b_ref[...]


def kernel(src, tgt, emb, pos_src, pos_tgt, Wc, bc):
    b, l = src.shape
    v, d = emb.shape
    m = 2 * b * l
    bm = l

    idx = jnp.concatenate([src.reshape(-1), tgt.reshape(-1)]).astype(jnp.int32)
    x = _gather_rows_sc(emb, idx)

    pos_cat = jnp.stack([pos_src[:l, :d], pos_tgt[:l, :d]])
    w_bf = Wc.astype(jnp.bfloat16)
    bc2 = bc.reshape(v, 1)

    x4 = x.reshape(2, b, l, d)
    bn = 1024
    nb = (v + bn - 1) // bn
    v_pad = nb * bn
    out = pl.pallas_call(
        functools.partial(_matmul_body, scale=math.sqrt(d), bn=bn),
        grid=(2, b, nb),
        in_specs=[
            pl.BlockSpec((1, 1, bm, d), lambda s, i, n: (s, i, 0, 0)),
            pl.BlockSpec((1, bm, d), lambda s, i, n: (s, 0, 0)),
            pl.BlockSpec(memory_space=pl.ANY),
            pl.BlockSpec((bn, 1), lambda s, i, n: (n, 0)),
        ],
        out_specs=pl.BlockSpec((1, 1, bn, bm), lambda s, i, n: (s, i, n, 0)),
        out_shape=jax.ShapeDtypeStruct((2, b, v, l), jnp.float32),
        scratch_shapes=[
            pltpu.VMEM((v_pad, d), jnp.bfloat16),
            pltpu.SemaphoreType.DMA,
        ],
        compiler_params=pltpu.CompilerParams(
            dimension_semantics=("arbitrary", "arbitrary", "arbitrary"),
        ),
    )(x4, pos_cat, w_bf, bc2)

    return jnp.swapaxes(out, 2, 3)


# speedup vs baseline: 1.8314x; 1.2148x over previous
import functools
import math

import jax
import jax.numpy as jnp
from jax import lax
from jax.experimental import pallas as pl
from jax.experimental.pallas import tpu as pltpu
from jax.experimental.pallas import tpu_sc as plsc


def _gather_rows_sc(table, idx):
    n_rows = idx.shape[0]
    _, d = table.shape
    info = plsc.get_sparse_core_info()
    nw = info.num_cores * info.num_subcores
    b_per_w = n_rows // nw
    mesh = plsc.VectorSubcoreMesh(core_axis_name="c", subcore_axis_name="s")

    @functools.partial(
        pl.kernel,
        mesh=mesh,
        out_type=jax.ShapeDtypeStruct((n_rows, d), table.dtype),
        scratch_types=[
            pltpu.VMEM((b_per_w,), jnp.int32),
            pltpu.VMEM((b_per_w, d), table.dtype),
            pltpu.SemaphoreType.DMA,
        ],
    )
    def gather_kernel(table_hbm, idx_hbm, out_hbm, idx_v, rows_v, sem):
        wid = lax.axis_index("s") * info.num_cores + lax.axis_index("c")
        base = wid * b_per_w
        pltpu.sync_copy(idx_hbm.at[pl.ds(base, b_per_w)], idx_v)
        pltpu.async_copy(table_hbm.at[idx_v], rows_v, sem).wait()
        pltpu.sync_copy(rows_v, out_hbm.at[pl.ds(base, b_per_w)])

    return gather_kernel(table, idx)


def _matmul_body(x_ref, pos_ref, w_hbm, b_ref, o_ref, w_vmem, sem, *, scale, bn):
    first = ((pl.program_id(0) == 0) & (pl.program_id(1) == 0)
             & (pl.program_id(2) == 0))

    @pl.when(first)
    def _():
        cp = pltpu.make_async_copy(w_hbm, w_vmem.at[pl.ds(0, w_hbm.shape[0])], sem)
        cp.start()
        cp.wait()

    n = pl.program_id(2)
    xb = (x_ref[0, 0] * scale + pos_ref[0]).astype(jnp.bfloat16)
    acc = lax.dot_general(
        w_vmem[pl.ds(n * bn, bn)], xb, (((1,), (1,)), ((), ())),
        preferred_element_type=jnp.float32,
    )
    o_ref[0, 0] = acc + b_ref[...]


def kernel(src, tgt, emb, pos_src, pos_tgt, Wc, bc):
    b, l = src.shape
    v, d = emb.shape
    m = 2 * b * l
    bm = l

    idx = jnp.concatenate([src.reshape(-1), tgt.reshape(-1)]).astype(jnp.int32)
    x = _gather_rows_sc(emb, idx)

    pos_cat = jnp.stack([pos_src[:l, :d], pos_tgt[:l, :d]])
    w_bf = Wc.astype(jnp.bfloat16)
    bc2 = bc.reshape(v, 1)

    x4 = x.reshape(2, b, l, d)
    bn = 2048
    nb = (v + bn - 1) // bn
    v_pad = nb * bn
    out = pl.pallas_call(
        functools.partial(_matmul_body, scale=math.sqrt(d), bn=bn),
        grid=(2, b, nb),
        in_specs=[
            pl.BlockSpec((1, 1, bm, d), lambda s, i, n: (s, i, 0, 0)),
            pl.BlockSpec((1, bm, d), lambda s, i, n: (s, 0, 0)),
            pl.BlockSpec(memory_space=pl.ANY),
            pl.BlockSpec((bn, 1), lambda s, i, n: (n, 0)),
        ],
        out_specs=pl.BlockSpec((1, 1, bn, bm), lambda s, i, n: (s, i, n, 0)),
        out_shape=jax.ShapeDtypeStruct((2, b, v, l), jnp.float32),
        scratch_shapes=[
            pltpu.VMEM((v_pad, d), jnp.bfloat16),
            pltpu.SemaphoreType.DMA,
        ],
        compiler_params=pltpu.CompilerParams(
            dimension_semantics=("arbitrary", "arbitrary", "arbitrary"),
        ),
    )(x4, pos_cat, w_bf, bc2)

    return jnp.swapaxes(out, 2, 3)


# speedup vs baseline: 2.2453x; 1.2260x over previous
import functools
import math

import jax
import jax.numpy as jnp
from jax import lax
from jax.experimental import pallas as pl
from jax.experimental.pallas import tpu as pltpu
from jax.experimental.pallas import tpu_sc as plsc


def _gather_rows_sc(table, idx):
    n_rows = idx.shape[0]
    _, d = table.shape
    info = plsc.get_sparse_core_info()
    nw = info.num_cores * info.num_subcores
    b_per_w = n_rows // nw
    mesh = plsc.VectorSubcoreMesh(core_axis_name="c", subcore_axis_name="s")

    @functools.partial(
        pl.kernel,
        mesh=mesh,
        out_type=jax.ShapeDtypeStruct((n_rows, d), table.dtype),
        scratch_types=[
            pltpu.VMEM((b_per_w,), jnp.int32),
            pltpu.VMEM((b_per_w, d), table.dtype),
            pltpu.SemaphoreType.DMA,
        ],
    )
    def gather_kernel(table_hbm, idx_hbm, out_hbm, idx_v, rows_v, sem):
        wid = lax.axis_index("s") * info.num_cores + lax.axis_index("c")
        base = wid * b_per_w
        pltpu.sync_copy(idx_hbm.at[pl.ds(base, b_per_w)], idx_v)
        pltpu.async_copy(table_hbm.at[idx_v], rows_v, sem).wait()
        pltpu.sync_copy(rows_v, out_hbm.at[pl.ds(base, b_per_w)])

    return gather_kernel(table, idx)


def _matmul_body(x_ref, pos_ref, w_hbm, b_ref, o_ref, w_vmem, sem, *, scale, bn):
    first = ((pl.program_id(0) == 0) & (pl.program_id(1) == 0)
             & (pl.program_id(2) == 0))

    @pl.when(first)
    def _():
        cp = pltpu.make_async_copy(w_hbm, w_vmem.at[pl.ds(0, w_hbm.shape[0])], sem)
        cp.start()
        cp.wait()

    n = pl.program_id(2)
    xb = (x_ref[0, 0] * scale + pos_ref[0]).astype(jnp.bfloat16)
    acc = lax.dot_general(
        w_vmem[pl.ds(n * bn, bn)], xb, (((1,), (1,)), ((), ())),
        preferred_element_type=jnp.float32,
    )
    o_ref[0, 0] = acc + b_ref[...]


def kernel(src, tgt, emb, pos_src, pos_tgt, Wc, bc):
    b, l = src.shape
    v, d = emb.shape
    m = 2 * b * l
    bm = l

    idx = jnp.concatenate([src.reshape(-1), tgt.reshape(-1)]).astype(jnp.int32)
    x = _gather_rows_sc(emb, idx)

    pos_cat = jnp.stack([pos_src[:l, :d], pos_tgt[:l, :d]])
    w_bf = Wc.astype(jnp.bfloat16)
    bc2 = bc.reshape(v, 1)

    x4 = x.reshape(2, b, l, d)
    bn = 4096
    nb = (v + bn - 1) // bn
    v_pad = nb * bn
    out = pl.pallas_call(
        functools.partial(_matmul_body, scale=math.sqrt(d), bn=bn),
        grid=(2, b, nb),
        in_specs=[
            pl.BlockSpec((1, 1, bm, d), lambda s, i, n: (s, i, 0, 0)),
            pl.BlockSpec((1, bm, d), lambda s, i, n: (s, 0, 0)),
            pl.BlockSpec(memory_space=pl.ANY),
            pl.BlockSpec((bn, 1), lambda s, i, n: (n, 0)),
        ],
        out_specs=pl.BlockSpec((1, 1, bn, bm), lambda s, i, n: (s, i, n, 0)),
        out_shape=jax.ShapeDtypeStruct((2, b, v, l), jnp.float32),
        scratch_shapes=[
            pltpu.VMEM((v_pad, d), jnp.bfloat16),
            pltpu.SemaphoreType.DMA,
        ],
        compiler_params=pltpu.CompilerParams(
            dimension_semantics=("arbitrary", "arbitrary", "arbitrary"),
        ),
    )(x4, pos_cat, w_bf, bc2)

    return jnp.swapaxes(out, 2, 3)


# speedup vs baseline: 2.8750x; 1.2805x over previous
import functools
import math

import jax
import jax.numpy as jnp
from jax import lax
from jax.experimental import pallas as pl
from jax.experimental.pallas import tpu as pltpu
from jax.experimental.pallas import tpu_sc as plsc


def _gather_rows_sc(table, idx):
    n_rows = idx.shape[0]
    _, d = table.shape
    info = plsc.get_sparse_core_info()
    nw = info.num_cores * info.num_subcores
    b_per_w = n_rows // nw
    mesh = plsc.VectorSubcoreMesh(core_axis_name="c", subcore_axis_name="s")

    @functools.partial(
        pl.kernel,
        mesh=mesh,
        out_type=jax.ShapeDtypeStruct((n_rows, d), table.dtype),
        scratch_types=[
            pltpu.VMEM((b_per_w,), jnp.int32),
            pltpu.VMEM((b_per_w, d), table.dtype),
            pltpu.SemaphoreType.DMA,
        ],
    )
    def gather_kernel(table_hbm, idx_hbm, out_hbm, idx_v, rows_v, sem):
        wid = lax.axis_index("s") * info.num_cores + lax.axis_index("c")
        base = wid * b_per_w
        pltpu.sync_copy(idx_hbm.at[pl.ds(base, b_per_w)], idx_v)
        pltpu.async_copy(table_hbm.at[idx_v], rows_v, sem).wait()
        pltpu.sync_copy(rows_v, out_hbm.at[pl.ds(base, b_per_w)])

    return gather_kernel(table, idx)


def _matmul_body(x_ref, pos_ref, w_hbm, b_ref, o_ref, w_vmem, sem, *, scale, bn):
    first = ((pl.program_id(0) == 0) & (pl.program_id(1) == 0)
             & (pl.program_id(2) == 0))

    @pl.when(first)
    def _():
        cp = pltpu.make_async_copy(w_hbm, w_vmem.at[pl.ds(0, w_hbm.shape[0])], sem)
        cp.start()
        cp.wait()

    n = pl.program_id(2)
    xb = (x_ref[0, 0] * scale + pos_ref[0]).astype(jnp.bfloat16)
    acc = lax.dot_general(
        w_vmem[pl.ds(n * bn, bn)], xb, (((1,), (1,)), ((), ())),
        preferred_element_type=jnp.float32,
    )
    o_ref[0, 0] = acc + b_ref[...]


def kernel(src, tgt, emb, pos_src, pos_tgt, Wc, bc):
    b, l = src.shape
    v, d = emb.shape
    m = 2 * b * l
    bm = l

    idx = jnp.concatenate([src.reshape(-1), tgt.reshape(-1)]).astype(jnp.int32)
    x = _gather_rows_sc(emb, idx)

    pos_cat = jnp.stack([pos_src[:l, :d], pos_tgt[:l, :d]])
    w_bf = Wc.astype(jnp.bfloat16)
    bc2 = bc.reshape(v, 1)

    x4 = x.reshape(2, b, l, d)
    bn = 8000
    nb = (v + bn - 1) // bn
    v_pad = nb * bn
    out = pl.pallas_call(
        functools.partial(_matmul_body, scale=math.sqrt(d), bn=bn),
        grid=(2, b, nb),
        in_specs=[
            pl.BlockSpec((1, 1, bm, d), lambda s, i, n: (s, i, 0, 0)),
            pl.BlockSpec((1, bm, d), lambda s, i, n: (s, 0, 0)),
            pl.BlockSpec(memory_space=pl.ANY),
            pl.BlockSpec((bn, 1), lambda s, i, n: (n, 0)),
        ],
        out_specs=pl.BlockSpec((1, 1, bn, bm), lambda s, i, n: (s, i, n, 0)),
        out_shape=jax.ShapeDtypeStruct((2, b, v, l), jnp.float32),
        scratch_shapes=[
            pltpu.VMEM((v_pad, d), jnp.bfloat16),
            pltpu.SemaphoreType.DMA,
        ],
        compiler_params=pltpu.CompilerParams(
            dimension_semantics=("arbitrary", "arbitrary", "arbitrary"),
        ),
    )(x4, pos_cat, w_bf, bc2)

    return jnp.swapaxes(out, 2, 3)


# speedup vs baseline: 3.0305x; 1.0541x over previous
import functools
import math

import jax
import jax.numpy as jnp
from jax import lax
from jax.experimental import pallas as pl
from jax.experimental.pallas import tpu as pltpu
from jax.experimental.pallas import tpu_sc as plsc

_GATHER_CHUNKS = 4


def _gather_rows_sc(table, idx):
    n_rows = idx.shape[0]
    _, d = table.shape
    info = plsc.get_sparse_core_info()
    nw = info.num_cores * info.num_subcores
    b_per_w = n_rows // nw
    rpc = b_per_w // _GATHER_CHUNKS
    mesh = plsc.VectorSubcoreMesh(core_axis_name="c", subcore_axis_name="s")

    @functools.partial(
        pl.kernel,
        mesh=mesh,
        out_type=jax.ShapeDtypeStruct((n_rows, d), table.dtype),
        scratch_types=[
            pltpu.VMEM((b_per_w,), jnp.int32),
            pltpu.VMEM((rpc, d), table.dtype),
            pltpu.VMEM((rpc, d), table.dtype),
            pltpu.SemaphoreType.DMA,
            pltpu.SemaphoreType.DMA,
            pltpu.SemaphoreType.DMA,
            pltpu.SemaphoreType.DMA,
        ],
    )
    def gather_kernel(table_hbm, idx_hbm, out_hbm, idx_v, r0, r1,
                      gs0, gs1, ws0, ws1):
        wid = lax.axis_index("s") * info.num_cores + lax.axis_index("c")
        base = wid * b_per_w
        pltpu.sync_copy(idx_hbm.at[pl.ds(base, b_per_w)], idx_v)
        bufs = (r0, r1)
        gsems = (gs0, gs1)
        wsems = (ws0, ws1)

        def gather_cp(c):
            return pltpu.make_async_copy(
                table_hbm.at[idx_v.at[pl.ds(c * rpc, rpc)]],
                bufs[c % 2], gsems[c % 2])

        def write_cp(c):
            return pltpu.make_async_copy(
                bufs[c % 2], out_hbm.at[pl.ds(base + c * rpc, rpc)],
                wsems[c % 2])

        gather_cp(0).start()
        for c in range(_GATHER_CHUNKS):
            gather_cp(c).wait()
            write_cp(c).start()
            if c + 1 < _GATHER_CHUNKS:
                if c >= 1:
                    write_cp(c - 1).wait()
                gather_cp(c + 1).start()
        write_cp(_GATHER_CHUNKS - 2).wait()
        write_cp(_GATHER_CHUNKS - 1).wait()

    return gather_kernel(table, idx)


def _matmul_body(x_ref, pos_ref, w_hbm, b_ref, o_ref, w32, wbf, sem, *,
                 scale):
    first = (pl.program_id(0) == 0) & (pl.program_id(1) == 0)

    @pl.when(first)
    def _():
        cp = pltpu.make_async_copy(w_hbm, w32, sem)
        cp.start()
        cp.wait()
        wbf[...] = w32[...].astype(jnp.bfloat16)

    xb = (x_ref[0, 0] * scale + pos_ref[0]).astype(jnp.bfloat16)
    acc = lax.dot_general(
        wbf[...], xb, (((1,), (1,)), ((), ())),
        preferred_element_type=jnp.float32,
    )
    o_ref[0, 0] = acc + b_ref[...]


def kernel(src, tgt, emb, pos_src, pos_tgt, Wc, bc):
    b, l = src.shape
    v, d = emb.shape
    bm = l

    idx = jnp.concatenate([src.reshape(-1), tgt.reshape(-1)]).astype(jnp.int32)
    x = _gather_rows_sc(emb, idx)

    pos_cat = jnp.stack([pos_src[:l, :d], pos_tgt[:l, :d]])
    bc2 = bc.reshape(v, 1)
    x4 = x.reshape(2, b, l, d)

    out = pl.pallas_call(
        functools.partial(_matmul_body, scale=math.sqrt(d)),
        grid=(2, b),
        in_specs=[
            pl.BlockSpec((1, 1, bm, d), lambda s, i: (s, i, 0, 0)),
            pl.BlockSpec((1, bm, d), lambda s, i: (s, 0, 0)),
            pl.BlockSpec(memory_space=pl.ANY),
            pl.BlockSpec((v, 1), lambda s, i: (0, 0)),
        ],
        out_specs=pl.BlockSpec((1, 1, v, bm), lambda s, i: (s, i, 0, 0)),
        out_shape=jax.ShapeDtypeStruct((2, b, v, l), jnp.float32),
        scratch_shapes=[
            pltpu.VMEM((v, d), jnp.float32),
            pltpu.VMEM((v, d), jnp.bfloat16),
            pltpu.SemaphoreType.DMA,
        ],
        compiler_params=pltpu.CompilerParams(
            dimension_semantics=("arbitrary", "arbitrary"),
        ),
    )(x4, pos_cat, Wc, bc2)

    return jnp.swapaxes(out, 2, 3)
